# baseline (speedup 1.0000x reference)
import functools

import jax
import jax.numpy as jnp
from jax import lax
from jax.experimental import pallas as pl
from jax.experimental.pallas import tpu as pltpu
from jax.experimental.pallas import tpu_sc as plsc

_E = 16
_K = 2
_D = 1024
_F = 2048
_T = 4096
_CAP = 640
_R = (_E + 1) * _CAP
_TRASH = _E * _CAP

_NC, _NS = 2, 16
_NW = _NC * _NS
_TPW = _T // _NW


def _route_body(x_ref, wg_ref, s0_ref, s1_ref, w0_ref, w1_ref, xb_ref):
    x = x_ref[...]
    wg = wg_ref[...]
    xb_ref[...] = x.astype(jnp.bfloat16)
    logits = jnp.dot(x, wg, preferred_element_type=jnp.float32)
    m = jnp.max(logits, axis=-1, keepdims=True)
    ex = jnp.exp(logits - m)
    gates = ex / jnp.sum(ex, axis=-1, keepdims=True)

    eidx = lax.broadcasted_iota(jnp.int32, (_T, _E), 1)
    m0 = jnp.max(gates, axis=-1, keepdims=True)
    i0 = jnp.min(jnp.where(gates == m0, eidx, _E), axis=-1, keepdims=True)
    oh0 = eidx == i0
    g2 = jnp.where(oh0, -jnp.inf, gates)
    m1 = jnp.max(g2, axis=-1, keepdims=True)
    i1 = jnp.min(jnp.where(g2 == m1, eidx, _E), axis=-1, keepdims=True)
    oh1 = eidx == i1

    ohf0 = oh0.astype(jnp.float32)
    ohf1 = oh1.astype(jnp.float32)

    C = 512
    rr = lax.broadcasted_iota(jnp.int32, (C, C), 0)
    cc = lax.broadcasted_iota(jnp.int32, (C, C), 1)
    tri = (cc <= rr).astype(jnp.float32)

    def chunked_cumsum(ohf):
        outs = []
        carry = jnp.zeros((1, _E), jnp.float32)
        for i in range(_T // C):
            blk = ohf[i * C:(i + 1) * C, :]
            cs = jnp.dot(tri, blk, preferred_element_type=jnp.float32) + carry
            outs.append(cs)
            carry = cs[C - 1:C, :]
        return jnp.concatenate(outs, axis=0)

    cum0 = chunked_cumsum(ohf0)
    cnt0 = cum0[_T - 1:_T, :]
    cum1 = chunked_cumsum(ohf1)

    pos0 = jnp.sum(cum0 * ohf0, axis=-1, keepdims=True) - 1.0
    off1 = jnp.sum(cnt0 * ohf1, axis=-1, keepdims=True)
    pos1 = jnp.sum(cum1 * ohf1, axis=-1, keepdims=True) - 1.0 + off1

    p0 = pos0.astype(jnp.int32)
    p1 = pos1.astype(jnp.int32)
    ok0 = p0 < _CAP
    ok1 = p1 < _CAP
    s0_ref[...] = jnp.where(ok0, i0 * _CAP + p0, _TRASH)
    s1_ref[...] = jnp.where(ok1, i1 * _CAP + p1, _TRASH)
    ssum = m0 + m1
    w0_ref[...] = jnp.broadcast_to(jnp.where(ok0, m0 / ssum, 0.0), (_T, 16))
    w1_ref[...] = jnp.broadcast_to(jnp.where(ok1, m1 / ssum, 0.0), (_T, 16))


def _route(x, wg, interpret=False):
    return pl.pallas_call(
        _route_body,
        out_shape=(
            jax.ShapeDtypeStruct((_T, 1), jnp.int32),
            jax.ShapeDtypeStruct((_T, 1), jnp.int32),
            jax.ShapeDtypeStruct((_T, 16), jnp.float32),
            jax.ShapeDtypeStruct((_T, 16), jnp.float32),
            jax.ShapeDtypeStruct((_T, _D), jnp.bfloat16),
        ),
        interpret=interpret,
    )(x, wg)


def _ffn_body(d_ref, w1_ref, w2_ref, o_ref):
    h = jnp.maximum(
        jnp.dot(d_ref[...], w1_ref[0].astype(jnp.bfloat16),
                preferred_element_type=jnp.float32), 0.0)
    o_ref[...] = jnp.dot(h.astype(jnp.bfloat16),
                         w2_ref[0].astype(jnp.bfloat16),
                         preferred_element_type=jnp.float32)


def _ffn(disp, w1, w2, interpret=False):
    nblk = _R // _CAP
    return pl.pallas_call(
        _ffn_body,
        grid=(nblk,),
        in_specs=[
            pl.BlockSpec((_CAP, _D), lambda b: (b, 0)),
            pl.BlockSpec((1, _D, _F), lambda b: (jnp.minimum(b, _E - 1), 0, 0)),
            pl.BlockSpec((1, _F, _D), lambda b: (jnp.minimum(b, _E - 1), 0, 0)),
        ],
        out_specs=pl.BlockSpec((_CAP, _D), lambda b: (b, 0)),
        out_shape=jax.ShapeDtypeStruct((_R, _D), jnp.float32),
        interpret=interpret,
    )(disp, w1, w2)


_DCH = 64

def _sc_mesh():
    return plsc.VectorSubcoreMesh(core_axis_name="c", subcore_axis_name="s",
                                  num_cores=_NC, num_subcores=_NS)


@functools.cache
def _make_dispatch():
    return pl.kernel(
        _dispatch_body,
        out_type=jax.ShapeDtypeStruct((_R, _D // 2), jnp.int32),
        mesh=_sc_mesh(),
        scratch_types=[
            pltpu.VMEM((2 * (_TPW // _DCH), _DCH), jnp.int32),
            pltpu.VMEM((_DCH, _D // 2), jnp.int32),
            pltpu.SemaphoreType.DMA,
        ],
    )


def _dispatch_body(x_hbm, s0_hbm, s1_hbm, disp_hbm, idx_v, rows_v, sem):
    wid = lax.axis_index("s") * _NC + lax.axis_index("c")
    base = wid * _TPW
    nch = _TPW // _DCH
    for ci in range(nch):
        pltpu.sync_copy(s0_hbm.at[pl.ds(base + ci * _DCH, _DCH)],
                        idx_v.at[ci])
        pltpu.sync_copy(s1_hbm.at[pl.ds(base + ci * _DCH, _DCH)],
                        idx_v.at[nch + ci])
    for ci in range(nch):
        pltpu.sync_copy(x_hbm.at[pl.ds(base + ci * _DCH, _DCH), :], rows_v)
        c0 = pltpu.async_copy(rows_v, disp_hbm.at[idx_v.at[ci]], sem)
        c1 = pltpu.async_copy(rows_v, disp_hbm.at[idx_v.at[nch + ci]], sem)
        c0.wait()
        c1.wait()


_CCH = 16


@functools.cache
def _make_combine():
    return pl.kernel(
        _combine_body,
        out_type=jax.ShapeDtypeStruct((_T, _D), jnp.float32),
        mesh=_sc_mesh(),
        scratch_types=[
            pltpu.VMEM((2, _TPW), jnp.int32),
            pltpu.VMEM((2, _TPW, 16), jnp.float32),
            pltpu.VMEM((2, _CCH, _D), jnp.float32),
            pltpu.VMEM((2, _CCH, _D), jnp.float32),
            pltpu.VMEM((_CCH, _D), jnp.float32),
            pltpu.SemaphoreType.DMA,
            pltpu.SemaphoreType.DMA,
        ],
    )


def _combine_body(eo_hbm, s0_hbm, s1_hbm, w0_hbm, w1_hbm, y_hbm,
                  idx_v, wv_v, r0_v, r1_v, out_v, sem0, sem1):
    wid = lax.axis_index("s") * _NC + lax.axis_index("c")
    base = wid * _TPW
    nch = _TPW // _CCH
    sems = (sem0, sem1)
    pltpu.sync_copy(s0_hbm.at[pl.ds(base, _TPW)], idx_v.at[0])
    pltpu.sync_copy(s1_hbm.at[pl.ds(base, _TPW)], idx_v.at[1])
    pltpu.sync_copy(w0_hbm.at[pl.ds(base, _TPW), :], wv_v.at[0])
    pltpu.sync_copy(w1_hbm.at[pl.ds(base, _TPW), :], wv_v.at[1])

    def fire(ci):
        pr = ci % 2
        g0 = pltpu.async_copy(
            eo_hbm.at[idx_v.at[0, pl.ds(ci * _CCH, _CCH)]],
            r0_v.at[pr], sems[pr])
        g1 = pltpu.async_copy(
            eo_hbm.at[idx_v.at[1, pl.ds(ci * _CCH, _CCH)]],
            r1_v.at[pr], sems[pr])
        return g0, g1

    pend = fire(0)
    for ci in range(nch):
        nxt = fire(ci + 1) if ci + 1 < nch else None
        pend[0].wait()
        pend[1].wait()
        pr = ci % 2

        def tok_body(j, _, ci=ci, pr=pr):
            w0v = wv_v[0, ci * _CCH + j, :]
            w1v = wv_v[1, ci * _CCH + j, :]
            for u in range(_D // 16):
                sl = slice(16 * u, 16 * u + 16)
                out_v[j, sl] = w0v * r0_v[pr, j, sl] + w1v * r1_v[pr, j, sl]
            return 0

        lax.fori_loop(0, _CCH, tok_body, 0)
        pltpu.sync_copy(out_v, y_hbm.at[pl.ds(base + ci * _CCH, _CCH), :])
        pend = nxt


@jax.jit
def kernel(x, wg, w1, w2):
    s0, s1, cw0, cw1, xb = _route(x, wg)
    s0 = s0.reshape(_T)
    s1 = s1.reshape(_T)
    xb32 = lax.bitcast_convert_type(xb.reshape(_T, _D // 2, 2), jnp.int32)
    disp32 = _make_dispatch()(xb32, s0, s1)
    disp = lax.bitcast_convert_type(disp32, jnp.bfloat16).reshape(_R, _D)
    eo = _ffn(disp, w1, w2)
    return _make_combine()(eo, s0, s1, cw0, cw1)


# speedup vs baseline: 2.3674x; 2.3674x over previous
import functools

import jax
import jax.numpy as jnp
from jax import lax
from jax.experimental import pallas as pl
from jax.experimental.pallas import tpu as pltpu
from jax.experimental.pallas import tpu_sc as plsc

_E = 16
_K = 2
_D = 1024
_F = 2048
_T = 4096
_CAP = 640
_R = (_E + 1) * _CAP
_TRASH = _E * _CAP

_NC, _NS = 2, 16
_NW = _NC * _NS
_TPW = _T // _NW


def _route_body(x_ref, wg_ref, s0_ref, s1_ref, w0_ref, w1_ref):
    x = x_ref[...]
    wg = wg_ref[...]
    logits = jnp.dot(x, wg, preferred_element_type=jnp.float32)
    m = jnp.max(logits, axis=-1, keepdims=True)
    ex = jnp.exp(logits - m)
    gates = ex / jnp.sum(ex, axis=-1, keepdims=True)

    eidx = lax.broadcasted_iota(jnp.int32, (_T, _E), 1)
    m0 = jnp.max(gates, axis=-1, keepdims=True)
    i0 = jnp.min(jnp.where(gates == m0, eidx, _E), axis=-1, keepdims=True)
    oh0 = eidx == i0
    g2 = jnp.where(oh0, -jnp.inf, gates)
    m1 = jnp.max(g2, axis=-1, keepdims=True)
    i1 = jnp.min(jnp.where(g2 == m1, eidx, _E), axis=-1, keepdims=True)
    oh1 = eidx == i1

    ohf0 = oh0.astype(jnp.float32)
    ohf1 = oh1.astype(jnp.float32)

    C = 512
    rr = lax.broadcasted_iota(jnp.int32, (C, C), 0)
    cc = lax.broadcasted_iota(jnp.int32, (C, C), 1)
    tri = (cc <= rr).astype(jnp.float32)

    def chunked_cumsum(ohf):
        outs = []
        carry = jnp.zeros((1, _E), jnp.float32)
        for i in range(_T // C):
            blk = ohf[i * C:(i + 1) * C, :]
            cs = jnp.dot(tri, blk, preferred_element_type=jnp.float32) + carry
            outs.append(cs)
            carry = cs[C - 1:C, :]
        return jnp.concatenate(outs, axis=0)

    cum0 = chunked_cumsum(ohf0)
    cnt0 = cum0[_T - 1:_T, :]
    cum1 = chunked_cumsum(ohf1)

    pos0 = jnp.sum(cum0 * ohf0, axis=-1, keepdims=True) - 1.0
    off1 = jnp.sum(cnt0 * ohf1, axis=-1, keepdims=True)
    pos1 = jnp.sum(cum1 * ohf1, axis=-1, keepdims=True) - 1.0 + off1

    p0 = pos0.astype(jnp.int32)
    p1 = pos1.astype(jnp.int32)
    ok0 = p0 < _CAP
    ok1 = p1 < _CAP
    s0_ref[...] = jnp.where(ok0, i0 * _CAP + p0, _TRASH)
    s1_ref[...] = jnp.where(ok1, i1 * _CAP + p1, _TRASH)
    ssum = m0 + m1
    w0_ref[...] = jnp.broadcast_to(jnp.where(ok0, m0 / ssum, 0.0), (_T, 16))
    w1_ref[...] = jnp.broadcast_to(jnp.where(ok1, m1 / ssum, 0.0), (_T, 16))


def _route(x, wg, interpret=False):
    return pl.pallas_call(
        _route_body,
        out_shape=(
            jax.ShapeDtypeStruct((_T, 1), jnp.int32),
            jax.ShapeDtypeStruct((_T, 1), jnp.int32),
            jax.ShapeDtypeStruct((_T, 16), jnp.float32),
            jax.ShapeDtypeStruct((_T, 16), jnp.float32),
        ),
        interpret=interpret,
    )(x, wg)


def _ffn_body(d_ref, w1_ref, w2_ref, o_ref):
    h = jnp.maximum(
        jnp.dot(d_ref[...], w1_ref[0], preferred_element_type=jnp.float32),
        0.0)
    o_ref[...] = jnp.dot(h, w2_ref[0], preferred_element_type=jnp.float32)


def _ffn(disp, w1, w2, interpret=False):
    nblk = _R // _CAP
    return pl.pallas_call(
        _ffn_body,
        grid=(nblk,),
        in_specs=[
            pl.BlockSpec((_CAP, _D), lambda b: (b, 0)),
            pl.BlockSpec((1, _D, _F), lambda b: (jnp.minimum(b, _E - 1), 0, 0)),
            pl.BlockSpec((1, _F, _D), lambda b: (jnp.minimum(b, _E - 1), 0, 0)),
        ],
        out_specs=pl.BlockSpec((_CAP, _D), lambda b: (b, 0)),
        out_shape=jax.ShapeDtypeStruct((_R, _D), jnp.float32),
        interpret=interpret,
    )(disp, w1, w2)


_DCH = 64

def _sc_mesh():
    return plsc.VectorSubcoreMesh(core_axis_name="c", subcore_axis_name="s",
                                  num_cores=_NC, num_subcores=_NS)


@functools.cache
def _make_dispatch():
    return pl.kernel(
        _dispatch_body,
        out_type=jax.ShapeDtypeStruct((_R, _D), jnp.float32),
        mesh=_sc_mesh(),
        scratch_types=[
            pltpu.VMEM((2 * (_TPW // _DCH), _DCH), jnp.int32),
            pltpu.VMEM((_DCH, _D), jnp.float32),
            pltpu.SemaphoreType.DMA,
        ],
    )


def _dispatch_body(x_hbm, s0_hbm, s1_hbm, disp_hbm, idx_v, rows_v, sem):
    wid = lax.axis_index("s") * _NC + lax.axis_index("c")
    base = wid * _TPW
    nch = _TPW // _DCH
    for ci in range(nch):
        pltpu.sync_copy(s0_hbm.at[pl.ds(base + ci * _DCH, _DCH)],
                        idx_v.at[ci])
        pltpu.sync_copy(s1_hbm.at[pl.ds(base + ci * _DCH, _DCH)],
                        idx_v.at[nch + ci])
    for ci in range(nch):
        pltpu.sync_copy(x_hbm.at[pl.ds(base + ci * _DCH, _DCH), :], rows_v)
        c0 = pltpu.async_copy(rows_v, disp_hbm.at[idx_v.at[ci]], sem)
        c1 = pltpu.async_copy(rows_v, disp_hbm.at[idx_v.at[nch + ci]], sem)
        c0.wait()
        c1.wait()


_CCH = 16


@functools.cache
def _make_combine():
    return pl.kernel(
        _combine_body,
        out_type=jax.ShapeDtypeStruct((_T, _D), jnp.float32),
        mesh=_sc_mesh(),
        scratch_types=[
            pltpu.VMEM((2, _TPW), jnp.int32),
            pltpu.VMEM((2, _TPW, 16), jnp.float32),
            pltpu.VMEM((2, _CCH, _D), jnp.float32),
            pltpu.VMEM((2, _CCH, _D), jnp.float32),
            pltpu.VMEM((_CCH, _D), jnp.float32),
            pltpu.SemaphoreType.DMA,
            pltpu.SemaphoreType.DMA,
        ],
    )


def _combine_body(eo_hbm, s0_hbm, s1_hbm, w0_hbm, w1_hbm, y_hbm,
                  idx_v, wv_v, r0_v, r1_v, out_v, sem0, sem1):
    wid = lax.axis_index("s") * _NC + lax.axis_index("c")
    base = wid * _TPW
    nch = _TPW // _CCH
    sems = (sem0, sem1)
    pltpu.sync_copy(s0_hbm.at[pl.ds(base, _TPW)], idx_v.at[0])
    pltpu.sync_copy(s1_hbm.at[pl.ds(base, _TPW)], idx_v.at[1])
    pltpu.sync_copy(w0_hbm.at[pl.ds(base, _TPW), :], wv_v.at[0])
    pltpu.sync_copy(w1_hbm.at[pl.ds(base, _TPW), :], wv_v.at[1])

    def fire(ci):
        pr = ci % 2
        g0 = pltpu.async_copy(
            eo_hbm.at[idx_v.at[0, pl.ds(ci * _CCH, _CCH)]],
            r0_v.at[pr], sems[pr])
        g1 = pltpu.async_copy(
            eo_hbm.at[idx_v.at[1, pl.ds(ci * _CCH, _CCH)]],
            r1_v.at[pr], sems[pr])
        return g0, g1

    pend = fire(0)
    for ci in range(nch):
        nxt = fire(ci + 1) if ci + 1 < nch else None
        pend[0].wait()
        pend[1].wait()
        pr = ci % 2

        def tok_body(j, _, ci=ci, pr=pr):
            w0v = wv_v[0, ci * _CCH + j, :]
            w1v = wv_v[1, ci * _CCH + j, :]
            for u in range(_D // 16):
                sl = slice(16 * u, 16 * u + 16)
                out_v[j, sl] = w0v * r0_v[pr, j, sl] + w1v * r1_v[pr, j, sl]
            return 0

        lax.fori_loop(0, _CCH, tok_body, 0)
        pltpu.sync_copy(out_v, y_hbm.at[pl.ds(base + ci * _CCH, _CCH), :])
        pend = nxt


@jax.jit
def kernel(x, wg, w1, w2):
    s0, s1, cw0, cw1 = _route(x, wg)
    s0 = s0.reshape(_T)
    s1 = s1.reshape(_T)
    disp = _make_dispatch()(x, s0, s1)
    eo = _ffn(disp, w1, w2)
    return _make_combine()(eo, s0, s1, cw0, cw1)
